# baseline (speedup 1.0000x reference)
import jax
import jax.numpy as jnp
from jax import lax
from jax.experimental import pallas as pl
from jax.experimental.pallas import tpu as pltpu
from jax.experimental.pallas import tpu_sc as plsc
import functools

N = 320000
D = 128
NSEG = 10000

NC = 2
NS = 16
CHUNK = 160
NCH = N // CHUNK
SEG_SPLIT = 5056
HALF0 = SEG_SPLIT
HALF1 = NSEG - SEG_SPLIT
ZROWS = 64


def _sc_segment_sum(rows, ids, split):
    mesh = plsc.VectorSubcoreMesh(core_axis_name="c", subcore_axis_name="s")

    @functools.partial(
        pl.kernel,
        out_type=jax.ShapeDtypeStruct((NSEG, D), jnp.float32),
        mesh=mesh,
        scratch_types=[
            pltpu.VMEM((CHUNK, D), jnp.float32),
            pltpu.VMEM((CHUNK, D), jnp.float32),
            pltpu.VMEM((CHUNK,), jnp.int32),
            pltpu.VMEM((CHUNK,), jnp.int32),
            pltpu.VMEM((ZROWS, D), jnp.float32),
            pltpu.VMEM((16,), jnp.int32),
            pltpu.VMEM_SHARED((NSEG, D), jnp.float32),
            pltpu.SemaphoreType.DMA,
            pltpu.SemaphoreType.DMA,
            pltpu.SemaphoreType.DMA,
            pltpu.SemaphoreType.DMA,
        ],
    )
    def body(rows_hbm, ids_hbm, split_hbm, out_hbm,
             rows_v0, rows_v1, idx_v0, idx_v1, zbuf, split_v, acc,
             rsem0, rsem1, isem0, isem1):
        cid = lax.axis_index("c")
        sid = lax.axis_index("s")
        rows_v = (rows_v0, rows_v1)
        idx_v = (idx_v0, idx_v1)
        rsem = (rsem0, rsem1)
        isem = (isem0, isem1)

        pltpu.sync_copy(split_hbm, split_v)
        r1 = split_v[...][0]
        ca = (r1 + CHUNK - 1) // CHUNK
        cb = r1 // CHUNK
        first = jnp.where(cid == 0, 0, cb)
        limit = jnp.where(cid == 0, ca, NCH - cb)
        nsteps = jnp.maximum(0, (limit - sid + NS - 1) // NS)

        def chunk_of(k):
            return first + sid + NS * k

        def start(k, b):
            off = pl.multiple_of(chunk_of(k) * CHUNK, CHUNK)
            pltpu.async_copy(ids_hbm.at[pl.ds(off, CHUNK)], idx_v[b], isem[b])
            pltpu.async_copy(rows_hbm.at[pl.ds(off, CHUNK)], rows_v[b], rsem[b])

        def wait(b):
            pltpu.make_async_copy(ids_hbm.at[pl.ds(0, CHUNK)], idx_v[b], isem[b]).wait()
            pltpu.make_async_copy(rows_hbm.at[pl.ds(0, CHUNK)], rows_v[b], rsem[b]).wait()

        def scatter(b):
            pltpu.sync_copy(rows_v[b], acc.at[idx_v[b]], add=True)

        @pl.when(nsteps > 0)
        def _():
            start(0, 0)

        zvec = jnp.zeros((16,), jnp.float32)

        def zstore(i, carry):
            r = i // 8
            c = lax.rem(i, 8) * 16
            zbuf[r, pl.ds(c, 16)] = zvec
            return carry

        lax.fori_loop(0, ZROWS * 8, zstore, 0)

        half_base = jnp.where(cid == 0, 0, SEG_SPLIT)
        half_len = jnp.where(cid == 0, HALF0, HALF1)
        zstart = sid * 320
        zlen = jnp.clip(half_len - zstart, 0, 320)
        nz = zlen // ZROWS
        zrem = zlen - nz * ZROWS

        def zcopy(i, carry):
            dst = pl.multiple_of(half_base + zstart + i * ZROWS, 8)
            pltpu.sync_copy(zbuf, acc.at[pl.ds(dst, ZROWS)])
            return carry

        lax.fori_loop(0, nz, zcopy, 0)

        @pl.when(zrem == 16)
        def _():
            dst = pl.multiple_of(half_base + zstart + nz * ZROWS, 8)
            pltpu.sync_copy(zbuf.at[pl.ds(0, 16)], acc.at[pl.ds(dst, 16)])

        plsc.subcore_barrier()

        def pair(p, carry):
            start(2 * p + 1, 1)
            wait(0)
            scatter(0)

            @pl.when(2 * p + 2 < nsteps)
            def _():
                start(2 * p + 2, 0)

            wait(1)
            scatter(1)
            return carry

        lax.fori_loop(0, nsteps // 2, pair, 0)

        @pl.when(lax.rem(nsteps, 2) == 1)
        def _():
            wait(0)
            scatter(0)

        plsc.subcore_barrier()

        wlen = jnp.clip(half_len - zstart, 0, 320)
        nw = wlen // ZROWS
        wrem = wlen - nw * ZROWS

        def wcopy(i, carry):
            off = pl.multiple_of(half_base + zstart + i * ZROWS, 8)
            pltpu.sync_copy(acc.at[pl.ds(off, ZROWS)], out_hbm.at[pl.ds(off, ZROWS)])
            return carry

        lax.fori_loop(0, nw, wcopy, 0)

        @pl.when(wrem == 16)
        def _():
            off = pl.multiple_of(half_base + zstart + nw * ZROWS, 8)
            pltpu.sync_copy(acc.at[pl.ds(off, 16)], out_hbm.at[pl.ds(off, 16)])

    return body(rows, ids, split)


def kernel(output, batch):
    ids = batch.astype(jnp.int32)
    r1 = jnp.searchsorted(ids, SEG_SPLIT).astype(jnp.int32)
    split = jnp.broadcast_to(r1, (16,))
    return _sc_segment_sum(output, ids, split).reshape(-1)


# speedup vs baseline: 1.2899x; 1.2899x over previous
import jax
import jax.numpy as jnp
from jax import lax
from jax.experimental import pallas as pl
from jax.experimental.pallas import tpu as pltpu
from jax.experimental.pallas import tpu_sc as plsc
import functools

N = 320000
D = 128
NSEG = 10000

NC = 2
NS = 16
CHUNK = 160
NCH = N // CHUNK
SEG_SPLIT = 5056
HALF0 = SEG_SPLIT
HALF1 = NSEG - SEG_SPLIT
ZROWS = 64


def _sc_segment_sum(rows, ids, split):
    mesh = plsc.VectorSubcoreMesh(core_axis_name="c", subcore_axis_name="s")

    @functools.partial(
        pl.kernel,
        out_type=jax.ShapeDtypeStruct((NSEG, D), jnp.float32),
        mesh=mesh,
        scratch_types=[
            pltpu.VMEM((CHUNK, D), jnp.float32),
            pltpu.VMEM((CHUNK, D), jnp.float32),
            pltpu.VMEM((CHUNK,), jnp.int32),
            pltpu.VMEM((CHUNK,), jnp.int32),
            pltpu.VMEM((ZROWS, D), jnp.float32),
            pltpu.VMEM((16,), jnp.int32),
            pltpu.VMEM_SHARED((NSEG, D), jnp.float32),
            pltpu.SemaphoreType.DMA,
            pltpu.SemaphoreType.DMA,
            pltpu.SemaphoreType.DMA,
            pltpu.SemaphoreType.DMA,
        ],
    )
    def body(rows_hbm, ids_hbm, split_hbm, out_hbm,
             rows_v0, rows_v1, idx_v0, idx_v1, zbuf, split_v, acc,
             rsem0, rsem1, isem0, isem1):
        cid = lax.axis_index("c")
        sid = lax.axis_index("s")
        rows_v = (rows_v0, rows_v1)
        idx_v = (idx_v0, idx_v1)
        rsem = (rsem0, rsem1)
        isem = (isem0, isem1)

        pltpu.sync_copy(split_hbm, split_v)
        r1 = split_v[...][0]
        ca = (r1 + CHUNK - 1) // CHUNK
        cb = r1 // CHUNK
        first = jnp.where(cid == 0, 0, cb)
        limit = jnp.where(cid == 0, ca, NCH - cb)
        nsteps = jnp.maximum(0, (limit - sid + NS - 1) // NS)

        def chunk_of(k):
            return first + sid + NS * k

        def start(k, b):
            off = pl.multiple_of(chunk_of(k) * CHUNK, CHUNK)
            pltpu.async_copy(ids_hbm.at[pl.ds(off, CHUNK)], idx_v[b], isem[b])
            pltpu.async_copy(rows_hbm.at[pl.ds(off, CHUNK)], rows_v[b], rsem[b])

        def wait(b):
            pltpu.make_async_copy(ids_hbm.at[pl.ds(0, CHUNK)], idx_v[b], isem[b]).wait()
            pltpu.make_async_copy(rows_hbm.at[pl.ds(0, CHUNK)], rows_v[b], rsem[b]).wait()

        def scatter(b):
            pltpu.sync_copy(rows_v[b], acc.at[idx_v[b]], add=True)

        @pl.when(nsteps > 0)
        def _():
            start(0, 0)

        zvec = jnp.zeros((16,), jnp.float32)

        def zstore(i, carry):
            r = i // 8
            c = lax.rem(i, 8) * 16
            zbuf[r, pl.ds(c, 16)] = zvec
            return carry

        lax.fori_loop(0, ZROWS * 8, zstore, 0)

        half_base = jnp.where(cid == 0, 0, SEG_SPLIT)
        half_len = jnp.where(cid == 0, HALF0, HALF1)
        zstart = sid * 320
        zlen = jnp.clip(half_len - zstart, 0, 320)
        nz = zlen // ZROWS
        zrem = zlen - nz * ZROWS

        def zcopy(i, carry):
            dst = pl.multiple_of(half_base + zstart + i * ZROWS, 8)
            pltpu.sync_copy(zbuf, acc.at[pl.ds(dst, ZROWS)])
            return carry

        lax.fori_loop(0, nz, zcopy, 0)

        @pl.when(zrem == 16)
        def _():
            dst = pl.multiple_of(half_base + zstart + nz * ZROWS, 8)
            pltpu.sync_copy(zbuf.at[pl.ds(0, 16)], acc.at[pl.ds(dst, 16)])

        plsc.subcore_barrier()

        def pair(p, carry):
            start(2 * p + 1, 1)
            wait(0)
            scatter(0)

            @pl.when(2 * p + 2 < nsteps)
            def _():
                start(2 * p + 2, 0)

            wait(1)
            scatter(1)
            return carry

        lax.fori_loop(0, nsteps // 2, pair, 0)

        @pl.when(lax.rem(nsteps, 2) == 1)
        def _():
            wait(0)
            scatter(0)

        plsc.subcore_barrier()

        wlen = jnp.clip(half_len - zstart, 0, 320)
        nw = wlen // ZROWS
        wrem = wlen - nw * ZROWS

        def wcopy(i, carry):
            off = pl.multiple_of(half_base + zstart + i * ZROWS, 8)
            pltpu.sync_copy(acc.at[pl.ds(off, ZROWS)], out_hbm.at[pl.ds(off, ZROWS)])
            return carry

        lax.fori_loop(0, nw, wcopy, 0)

        @pl.when(wrem == 16)
        def _():
            off = pl.multiple_of(half_base + zstart + nw * ZROWS, 8)
            pltpu.sync_copy(acc.at[pl.ds(off, 16)], out_hbm.at[pl.ds(off, 16)])

    return body(rows, ids, split)


def kernel(output, batch):
    ids = batch.astype(jnp.int32)
    r1 = jnp.sum((ids < SEG_SPLIT).astype(jnp.int32)).astype(jnp.int32)
    split = jnp.broadcast_to(r1, (16,))
    return _sc_segment_sum(output, ids, split).reshape(-1)


# speedup vs baseline: 1.2937x; 1.0029x over previous
import jax
import jax.numpy as jnp
from jax import lax
from jax.experimental import pallas as pl
from jax.experimental.pallas import tpu as pltpu
from jax.experimental.pallas import tpu_sc as plsc
import functools

N = 320000
D = 128
NSEG = 10000

NC = 2
NS = 16
CHUNK = 320
NCH = N // CHUNK
SEG_SPLIT = 5056
HALF0 = SEG_SPLIT
HALF1 = NSEG - SEG_SPLIT
ACC_ROWS = 5064
ZROWS = 56


def _sc_segment_sum(rows, ids, split):
    mesh = plsc.VectorSubcoreMesh(core_axis_name="c", subcore_axis_name="s")

    @functools.partial(
        pl.kernel,
        out_type=jax.ShapeDtypeStruct((NSEG, D), jnp.float32),
        mesh=mesh,
        scratch_types=[
            pltpu.VMEM((CHUNK, D), jnp.float32),
            pltpu.VMEM((CHUNK, D), jnp.float32),
            pltpu.VMEM((CHUNK,), jnp.int32),
            pltpu.VMEM((CHUNK,), jnp.int32),
            pltpu.VMEM((ZROWS, D), jnp.float32),
            pltpu.VMEM((16,), jnp.int32),
            pltpu.VMEM_SHARED((ACC_ROWS, D), jnp.float32),
            pltpu.SemaphoreType.DMA,
            pltpu.SemaphoreType.DMA,
            pltpu.SemaphoreType.DMA,
            pltpu.SemaphoreType.DMA,
        ],
    )
    def body(rows_hbm, ids_hbm, split_hbm, out_hbm,
             rows_v0, rows_v1, idx_v0, idx_v1, zbuf, split_v, acc,
             rsem0, rsem1, isem0, isem1):
        cid = lax.axis_index("c")
        sid = lax.axis_index("s")
        rows_v = (rows_v0, rows_v1)
        idx_v = (idx_v0, idx_v1)
        rsem = (rsem0, rsem1)
        isem = (isem0, isem1)

        pltpu.sync_copy(split_hbm, split_v)
        r1 = split_v[...][0]
        ca = (r1 + CHUNK - 1) // CHUNK
        cb = r1 // CHUNK
        first = jnp.where(cid == 0, 0, cb)
        limit = jnp.where(cid == 0, ca, NCH - cb)
        nsteps = jnp.maximum(0, (limit - sid + NS - 1) // NS)

        half_lo = jnp.where(cid == 0, 0, SEG_SPLIT)
        half_len = jnp.where(cid == 0, HALF0, HALF1)
        lo_v = jnp.full((16,), half_lo, jnp.int32)
        len_v = jnp.full((16,), half_len, jnp.int32)

        def chunk_of(k):
            return first + sid + NS * k

        def start(k, b):
            off = pl.multiple_of(chunk_of(k) * CHUNK, CHUNK)
            pltpu.async_copy(ids_hbm.at[pl.ds(off, CHUNK)], idx_v[b], isem[b])
            pltpu.async_copy(rows_hbm.at[pl.ds(off, CHUNK)], rows_v[b], rsem[b])

        def wait(b):
            pltpu.make_async_copy(ids_hbm.at[pl.ds(0, CHUNK)], idx_v[b], isem[b]).wait()
            pltpu.make_async_copy(rows_hbm.at[pl.ds(0, CHUNK)], rows_v[b], rsem[b]).wait()

        def localize(b):
            for j in range(CHUNK // 16):
                v = idx_v[b][pl.ds(j * 16, 16)]
                loc = v - lo_v
                ok = (loc >= 0) & (loc < len_v)
                idx_v[b][pl.ds(j * 16, 16)] = jnp.where(ok, loc, len_v)

        def scatter(b):
            pltpu.sync_copy(rows_v[b], acc.at[idx_v[b]], add=True)

        @pl.when(nsteps > 0)
        def _():
            start(0, 0)

        zvec = jnp.zeros((16,), jnp.float32)

        def zstore(i, carry):
            r = i // 8
            c = lax.rem(i, 8) * 16
            zbuf[r, pl.ds(c, 16)] = zvec
            return carry

        lax.fori_loop(0, ZROWS * 8, zstore, 0)

        zstart = sid * 320
        zlen = jnp.clip(ACC_ROWS - zstart, 0, 320)
        nz = zlen // ZROWS

        def zcopy(i, carry):
            dst = pl.multiple_of(zstart + i * ZROWS, 8)
            pltpu.sync_copy(zbuf, acc.at[pl.ds(dst, ZROWS)])
            return carry

        lax.fori_loop(0, nz, zcopy, 0)
        zdst = pl.multiple_of(zstart + nz * ZROWS, 8)
        pltpu.sync_copy(zbuf.at[pl.ds(0, 40)], acc.at[pl.ds(zdst, 40)])

        plsc.subcore_barrier()

        def pair(p, carry):
            start(2 * p + 1, 1)
            wait(0)
            localize(0)
            scatter(0)

            @pl.when(2 * p + 2 < nsteps)
            def _():
                start(2 * p + 2, 0)

            wait(1)
            localize(1)
            scatter(1)
            return carry

        lax.fori_loop(0, nsteps // 2, pair, 0)

        @pl.when(lax.rem(nsteps, 2) == 1)
        def _():
            wait(0)
            localize(0)
            scatter(0)

        plsc.subcore_barrier()

        obase = pl.multiple_of(sid * 320, 8)

        @pl.when(sid < NS - 1)
        def _():
            pltpu.sync_copy(acc.at[pl.ds(obase, 320)],
                            out_hbm.at[pl.ds(pl.multiple_of(half_lo + obase, 8), 320)])

        @pl.when((sid == NS - 1) & (cid == 0))
        def _():
            pltpu.sync_copy(acc.at[pl.ds(4800, 256)],
                            out_hbm.at[pl.ds(4800, 256)])

        @pl.when((sid == NS - 1) & (cid == 1))
        def _():
            pltpu.sync_copy(acc.at[pl.ds(4800, 144)],
                            out_hbm.at[pl.ds(SEG_SPLIT + 4800, 144)])

    return body(rows, ids, split)


def kernel(output, batch):
    ids = batch.astype(jnp.int32)
    r1 = jnp.sum((ids < SEG_SPLIT).astype(jnp.int32)).astype(jnp.int32)
    split = jnp.broadcast_to(r1, (16,))
    return _sc_segment_sum(output, ids, split).reshape(-1)
